# baseline (speedup 1.0000x reference)
import jax
import jax.numpy as jnp
from jax import lax
from jax.experimental import pallas as pl
from jax.experimental.pallas import tpu as pltpu
from jax.experimental.pallas import tpu_sc as plsc

N_FIELDS = 26
VOCAB = 100000
HIDDEN = 32
BATCH = 16384

NUM_CORES = 2
NUM_SUBCORES = 16
NW = NUM_CORES * NUM_SUBCORES
BPW = BATCH // NW
CHUNK = 64
NCHUNK = BPW // CHUNK
ROWS = CHUNK * N_FIELDS
IWIDTH = 128
NSTREAM = ROWS // IWIDTH
LANES = 16


def _body(x_hbm, tab_hbm, out_hbm, idx_v, rows_v, out_v, sem0, sem1):
    wid = lax.axis_index("s") * NUM_CORES + lax.axis_index("c")
    sems = [sem0, sem1]

    def load_and_fire(t, b):
        pltpu.sync_copy(x_hbm.at[wid, t], idx_v.at[b])
        for r in range(NSTREAM):
            for k in range(IWIDTH // LANES):
                f = (r * IWIDTH + k * LANES) // CHUNK
                sl = pl.ds(k * LANES, LANES)
                idx_v[b, r, sl] = idx_v[b, r, sl] + jnp.int32(f * VOCAB)
        return [
            pltpu.async_copy(
                tab_hbm.at[idx_v.at[b, r]],
                rows_v.at[b, pl.ds(r * IWIDTH, IWIDTH)],
                sems[b])
            for r in range(NSTREAM)
        ]

    def accumulate(b):
        def elem_body(c, carry):
            a = rows_v[b, c, 0:32]
            for i in range(1, N_FIELDS):
                a = a + rows_v[b, i * CHUNK + c, 0:32]
            out_v[b, c, 0:32] = a
            return carry

        lax.fori_loop(0, CHUNK, elem_body, 0)

    handles = load_and_fire(0, 0)
    for t in range(NCHUNK):
        b = t % 2
        nxt = None
        if t + 1 < NCHUNK:
            nxt = load_and_fire(t + 1, (t + 1) % 2)
        for h in handles:
            h.wait()
        accumulate(b)
        pltpu.sync_copy(out_v.at[b],
                        out_hbm.at[pl.ds(wid * BPW + t * CHUNK, CHUNK)])
        handles = nxt


_mesh = plsc.VectorSubcoreMesh(core_axis_name="c", subcore_axis_name="s")

_sc_call = pl.kernel(
    _body,
    out_type=jax.ShapeDtypeStruct((BATCH, HIDDEN), jnp.bfloat16),
    mesh=_mesh,
    scratch_types=[
        pltpu.VMEM((2, NSTREAM, IWIDTH), jnp.int32),
        pltpu.VMEM((2, ROWS, HIDDEN), jnp.bfloat16),
        pltpu.VMEM((2, CHUNK, HIDDEN), jnp.bfloat16),
        pltpu.SemaphoreType.DMA,
        pltpu.SemaphoreType.DMA,
    ],
    compiler_params=pltpu.CompilerParams(use_tc_tiling_on_sc=False),
)


def kernel(x, tables):
    x4 = (x.astype(jnp.int32)
          .reshape(N_FIELDS, NW, NCHUNK, CHUNK)
          .transpose(1, 2, 0, 3)
          .reshape(NW, NCHUNK, NSTREAM, IWIDTH))
    tab_flat = tables.reshape(N_FIELDS * VOCAB, HIDDEN).astype(jnp.bfloat16)
    return _sc_call(x4, tab_flat).astype(jnp.float32)


# speedup vs baseline: 1.1927x; 1.1927x over previous
import jax
import jax.numpy as jnp
from jax import lax
from jax.experimental import pallas as pl
from jax.experimental.pallas import tpu as pltpu
from jax.experimental.pallas import tpu_sc as plsc

N_FIELDS = 26
VOCAB = 100000
HIDDEN = 32
BATCH = 16384

NUM_CORES = 2
NUM_SUBCORES = 16
NW = NUM_CORES * NUM_SUBCORES
BPW = BATCH // NW
CHUNK = 64
NCHUNK = BPW // CHUNK
ROWS = CHUNK * N_FIELDS
IWIDTH = 128
NSTREAM = ROWS // IWIDTH
LANES = 16


def _body(x_hbm, tab_hbm, out_hbm, idx_v, rows_v, out_v, sem0, sem1):
    wid = lax.axis_index("s") * NUM_CORES + lax.axis_index("c")
    sems = [sem0, sem1]

    def load_and_fire(t, b):
        pltpu.sync_copy(x_hbm.at[wid, t], idx_v.at[b])
        for k in range(ROWS // LANES):
            f = (k * LANES) // CHUNK
            sl = pl.ds(k * LANES, LANES)
            idx_v[b, sl] = idx_v[b, sl] + jnp.int32(f * VOCAB)
        return pltpu.async_copy(tab_hbm.at[idx_v.at[b]], rows_v.at[b], sems[b])

    def accumulate(b):
        def elem_body(c, carry):
            a0 = rows_v[b, c, 0:16]
            a1 = rows_v[b, c, 16:32]
            for i in range(1, N_FIELDS):
                a0 = a0 + rows_v[b, i * CHUNK + c, 0:16]
                a1 = a1 + rows_v[b, i * CHUNK + c, 16:32]
            out_v[b, c, 0:16] = a0
            out_v[b, c, 16:32] = a1
            return carry

        lax.fori_loop(0, CHUNK, elem_body, 0)

    handle = load_and_fire(0, 0)
    for t in range(NCHUNK):
        b = t % 2
        nxt = None
        if t + 1 < NCHUNK:
            nxt = load_and_fire(t + 1, (t + 1) % 2)
        handle.wait()
        accumulate(b)
        pltpu.sync_copy(out_v.at[b],
                        out_hbm.at[pl.ds(wid * BPW + t * CHUNK, CHUNK)])
        handle = nxt


_mesh = plsc.VectorSubcoreMesh(core_axis_name="c", subcore_axis_name="s")

_sc_call = pl.kernel(
    _body,
    out_type=jax.ShapeDtypeStruct((BATCH, HIDDEN), jnp.float32),
    mesh=_mesh,
    scratch_types=[
        pltpu.VMEM((2, ROWS), jnp.int32),
        pltpu.VMEM((2, ROWS, HIDDEN), jnp.float32),
        pltpu.VMEM((2, CHUNK, HIDDEN), jnp.float32),
        pltpu.SemaphoreType.DMA,
        pltpu.SemaphoreType.DMA,
    ],
    compiler_params=pltpu.CompilerParams(use_tc_tiling_on_sc=False),
)


def kernel(x, tables):
    x4 = (x.astype(jnp.int32)
          .reshape(N_FIELDS, NW, NCHUNK, CHUNK)
          .transpose(1, 2, 0, 3)
          .reshape(NW, NCHUNK, ROWS))
    tab_flat = tables.reshape(N_FIELDS * VOCAB, HIDDEN)
    return _sc_call(x4, tab_flat)
